# Initial kernel scaffold; baseline (speedup 1.0000x reference)
import jax
import jax.numpy as jnp
from jax.experimental import pallas as pl


def kernel(x, edge_index, W, b):
    raise NotImplementedError("write your pallas kernel here")


# speedup vs baseline: 4.6545x; 4.6545x over previous
import functools

import jax
import jax.numpy as jnp
from jax import lax
from jax.experimental import pallas as pl
from jax.experimental.pallas import tpu as pltpu
from jax.experimental.pallas import tpu_sc as plsc

N_NODES = 10000
N_EDGES = 320000
D = 128

NC = 2
NS = 16
NW = NC * NS

EDGES_PER_TILE = N_EDGES // NW
CHUNK = 80
STEPS = EDGES_PER_TILE // CHUNK
ROWS_PER_TILE = 632
N_PAD = NS * ROWS_PER_TILE

DEG_W = 16

_mesh = plsc.VectorSubcoreMesh(
    core_axis_name="c", subcore_axis_name="s", num_cores=NC, num_subcores=NS
)


import dataclasses as _dc

_cp = pltpu.CompilerParams()
if "needs_layout_passes" in pltpu.CompilerParams.__dataclass_fields__:
    _cp = _dc.replace(_cp, needs_layout_passes=False)


def _deg_body(src_hbm, out_hbm, stage, hist_v, idx_v, row_v, acc_v, bcast_v):
    cid = lax.axis_index("c")
    sid = lax.axis_index("s")
    wid = cid * NS + sid

    @pl.loop(0, N_PAD // 16)
    def _(i):
        hist_v[pl.ds(i * 16, 16)] = jnp.zeros((16,), jnp.float32)

    @pl.loop(0, STEPS)
    def _(t):
        base = wid * EDGES_PER_TILE + t * CHUNK
        pltpu.sync_copy(src_hbm.at[pl.ds(base, CHUNK)], idx_v)
        for j in range(CHUNK // 16):
            iv = idx_v[pl.ds(j * 16, 16)]
            plsc.addupdate_scatter(hist_v, [iv], jnp.full((16,), 1.0, jnp.float32))

    pltpu.sync_copy(hist_v, stage.at[pl.ds(sid * N_PAD, N_PAD)])
    plsc.subcore_barrier()

    @pl.loop(0, ROWS_PER_TILE // 16)
    def _(k):
        acc_v[pl.ds(k * 16, 16)] = jnp.zeros((16,), jnp.float32)

    @pl.loop(0, NS)
    def _(j):
        pltpu.sync_copy(stage.at[pl.ds(j * N_PAD + sid * ROWS_PER_TILE, ROWS_PER_TILE)], row_v)

        @pl.loop(0, ROWS_PER_TILE // 16)
        def _(k):
            acc_v[pl.ds(k * 16, 16)] = acc_v[pl.ds(k * 16, 16)] + row_v[pl.ds(k * 16, 16)]

    @pl.loop(0, ROWS_PER_TILE)
    def _(i):
        v = plsc.load_gather(acc_v, [jnp.full((16,), i, jnp.int32)])
        bcast_v[i] = v

    pltpu.sync_copy(bcast_v, out_hbm.at[cid, pl.ds(sid * ROWS_PER_TILE, ROWS_PER_TILE)])


_deg_kernel = pl.kernel(
    _deg_body,
    out_type=jax.ShapeDtypeStruct((NC, N_PAD, DEG_W), jnp.float32),
    mesh=_mesh,
    scratch_types=[
        pltpu.VMEM_SHARED((NS * N_PAD,), jnp.float32),
        pltpu.VMEM((N_PAD,), jnp.float32),
        pltpu.VMEM((CHUNK,), jnp.int32),
        pltpu.VMEM((ROWS_PER_TILE,), jnp.float32),
        pltpu.VMEM((ROWS_PER_TILE,), jnp.float32),
        pltpu.VMEM((ROWS_PER_TILE, DEG_W), jnp.float32),
    ],
    compiler_params=_cp,
)


def _agg_body(yn_hbm, src_hbm, dst_hbm, zeros_hbm, out_hbm,
              acc, sidx_v, didx_v, rows_v):
    cid = lax.axis_index("c")
    sid = lax.axis_index("s")
    wid = cid * NS + sid

    pltpu.sync_copy(zeros_hbm, acc.at[pl.ds(sid * ROWS_PER_TILE, ROWS_PER_TILE)])
    plsc.subcore_barrier()

    @pl.loop(0, STEPS)
    def _(t):
        base = wid * EDGES_PER_TILE + t * CHUNK
        pltpu.sync_copy(src_hbm.at[pl.ds(base, CHUNK)], sidx_v)
        pltpu.sync_copy(dst_hbm.at[pl.ds(base, CHUNK)], didx_v)
        pltpu.sync_copy(yn_hbm.at[sidx_v], rows_v)
        pltpu.sync_copy(rows_v, acc.at[didx_v], add=True)

    plsc.subcore_barrier()
    pltpu.sync_copy(
        acc.at[pl.ds(sid * ROWS_PER_TILE, ROWS_PER_TILE)],
        out_hbm.at[cid, pl.ds(sid * ROWS_PER_TILE, ROWS_PER_TILE)],
    )


_agg_kernel = pl.kernel(
    _agg_body,
    out_type=jax.ShapeDtypeStruct((NC, N_PAD, D), jnp.float32),
    mesh=_mesh,
    scratch_types=[
        pltpu.VMEM_SHARED((N_PAD, D), jnp.float32),
        pltpu.VMEM((CHUNK,), jnp.int32),
        pltpu.VMEM((CHUNK,), jnp.int32),
        pltpu.VMEM((CHUNK, D), jnp.float32),
    ],
)


def _matmul_body(x_ref, w_ref, o_ref):
    o_ref[...] = lax.dot_general(
        x_ref[...], w_ref[...], (((1,), (1,)), ((), ())),
        preferred_element_type=jnp.float32,
    )


def _norm_body(y_ref, degp_ref, o_ref):
    deg = degp_ref[0] + degp_ref[1]
    inv = lax.rsqrt(deg[:, 0:1] + 1.0)
    o_ref[...] = y_ref[...] * inv


def _final_body(a_ref, yn_ref, b_ref, o_ref):
    o_ref[...] = a_ref[0] + a_ref[1] + yn_ref[...] + b_ref[...]


_RB = 1000
_GRID = N_NODES // _RB


def kernel(x, edge_index, W, b):
    src = edge_index[0]
    dst = edge_index[1]

    y = pl.pallas_call(
        _matmul_body,
        grid=(_GRID,),
        in_specs=[
            pl.BlockSpec((_RB, D), lambda i: (i, 0)),
            pl.BlockSpec((D, D), lambda i: (0, 0)),
        ],
        out_specs=pl.BlockSpec((_RB, D), lambda i: (i, 0)),
        out_shape=jax.ShapeDtypeStruct((N_NODES, D), jnp.float32),
    )(x, W)

    degp = _deg_kernel(src)

    yn = pl.pallas_call(
        _norm_body,
        grid=(_GRID,),
        in_specs=[
            pl.BlockSpec((_RB, D), lambda i: (i, 0)),
            pl.BlockSpec((NC, _RB, DEG_W), lambda i: (0, i, 0)),
        ],
        out_specs=pl.BlockSpec((_RB, D), lambda i: (i, 0)),
        out_shape=jax.ShapeDtypeStruct((N_NODES, D), jnp.float32),
    )(y, degp)

    zeros_rows = jnp.zeros((ROWS_PER_TILE, D), jnp.float32)
    aggp = _agg_kernel(yn, src, dst, zeros_rows)

    out = pl.pallas_call(
        _final_body,
        grid=(_GRID,),
        in_specs=[
            pl.BlockSpec((NC, _RB, D), lambda i: (0, i, 0)),
            pl.BlockSpec((_RB, D), lambda i: (i, 0)),
            pl.BlockSpec((1, D), lambda i: (0, 0)),
        ],
        out_specs=pl.BlockSpec((_RB, D), lambda i: (i, 0)),
        out_shape=jax.ShapeDtypeStruct((N_NODES, D), jnp.float32),
    )(aggp, yn, b.reshape(1, D))

    return out


# speedup vs baseline: 10.1846x; 2.1881x over previous
import functools

import jax
import jax.numpy as jnp
from jax import lax
from jax.experimental import pallas as pl
from jax.experimental.pallas import tpu as pltpu
from jax.experimental.pallas import tpu_sc as plsc

N_NODES = 10000
N_EDGES = 320000
D = 128

NC = 2
NS = 16
NW = NC * NS

EDGES_PER_TILE = N_EDGES // NW
CHUNK = 80
STEPS = EDGES_PER_TILE // CHUNK
ROWS_PER_TILE = 632
N_PAD = NS * ROWS_PER_TILE

DEG_W = 16

_mesh = plsc.VectorSubcoreMesh(
    core_axis_name="c", subcore_axis_name="s", num_cores=NC, num_subcores=NS
)


import dataclasses as _dc

_cp = pltpu.CompilerParams()
if "needs_layout_passes" in pltpu.CompilerParams.__dataclass_fields__:
    _cp = _dc.replace(_cp, needs_layout_passes=False)


def _deg_body(src_hbm, out_hbm, stage, hist_v, idx_v, row_v, acc_v, bcast_v):
    cid = lax.axis_index("c")
    sid = lax.axis_index("s")
    wid = cid * NS + sid

    @pl.loop(0, N_PAD // 16)
    def _(i):
        hist_v[pl.ds(i * 16, 16)] = jnp.zeros((16,), jnp.float32)

    pltpu.sync_copy(src_hbm.at[pl.ds(wid * EDGES_PER_TILE, EDGES_PER_TILE)], idx_v)

    @pl.loop(0, EDGES_PER_TILE // 16)
    def _(i):
        iv = idx_v[pl.ds(i * 16, 16)]
        plsc.addupdate_scatter(hist_v, [iv], jnp.full((16,), 1.0, jnp.float32))

    pltpu.sync_copy(hist_v, stage.at[pl.ds(sid * N_PAD, N_PAD)])
    plsc.subcore_barrier()

    @pl.loop(0, ROWS_PER_TILE // 16)
    def _(k):
        acc_v[pl.ds(k * 16, 16)] = jnp.zeros((16,), jnp.float32)

    @pl.loop(0, NS)
    def _(j):
        pltpu.sync_copy(stage.at[pl.ds(j * N_PAD + sid * ROWS_PER_TILE, ROWS_PER_TILE)], row_v)

        @pl.loop(0, ROWS_PER_TILE // 16)
        def _(k):
            acc_v[pl.ds(k * 16, 16)] = acc_v[pl.ds(k * 16, 16)] + row_v[pl.ds(k * 16, 16)]

    @pl.loop(0, ROWS_PER_TILE)
    def _(i):
        v = plsc.load_gather(acc_v, [jnp.full((16,), i, jnp.int32)])
        bcast_v[i] = v

    pltpu.sync_copy(bcast_v, out_hbm.at[cid, pl.ds(sid * ROWS_PER_TILE, ROWS_PER_TILE)])


_deg_kernel = pl.kernel(
    _deg_body,
    out_type=jax.ShapeDtypeStruct((NC, N_PAD, DEG_W), jnp.float32),
    mesh=_mesh,
    scratch_types=[
        pltpu.VMEM_SHARED((NS * N_PAD,), jnp.float32),
        pltpu.VMEM((N_PAD,), jnp.float32),
        pltpu.VMEM((EDGES_PER_TILE,), jnp.int32),
        pltpu.VMEM((ROWS_PER_TILE,), jnp.float32),
        pltpu.VMEM((ROWS_PER_TILE,), jnp.float32),
        pltpu.VMEM((ROWS_PER_TILE, DEG_W), jnp.float32),
    ],
    compiler_params=_cp,
)


AG_CH = 64
AG_STEPS = 156
AG_TAIL = EDGES_PER_TILE - AG_CH * AG_STEPS
NSLOT = 3
AG_ITERS = AG_STEPS // NSLOT


def _agg_body(yn_hbm, src_hbm, dst_hbm, zeros_hbm, out_hbm,
              acc, sidx_v, didx_v, rows0, rows1, rows2, sems):
    cid = lax.axis_index("c")
    sid = lax.axis_index("s")
    wid = cid * NS + sid
    rows = [rows0, rows1, rows2]
    ebase = wid * EDGES_PER_TILE

    z = pltpu.async_copy(
        zeros_hbm, acc.at[pl.ds(sid * ROWS_PER_TILE, ROWS_PER_TILE)],
        sems.at[2 * NSLOT])
    si = pltpu.async_copy(
        src_hbm.at[pl.ds(ebase, EDGES_PER_TILE)], sidx_v, sems.at[2 * NSLOT + 1])
    di = pltpu.async_copy(
        dst_hbm.at[pl.ds(ebase, EDGES_PER_TILE)], didx_v, sems.at[2 * NSLOT + 2])
    z.wait()
    si.wait()
    di.wait()
    plsc.subcore_barrier()

    def gather_slice(t):
        return yn_hbm.at[sidx_v.at[pl.ds(t * AG_CH, AG_CH)]]

    def scatter_slice(t):
        return acc.at[didx_v.at[pl.ds(t * AG_CH, AG_CH)]]

    for b in range(NSLOT):
        pltpu.async_copy(gather_slice(b), rows[b], sems.at[b])

    @pl.loop(0, AG_ITERS - 1)
    def _(k):
        T = k * NSLOT
        sds = []
        for b in range(NSLOT):
            t = T + b
            pltpu.make_async_copy(gather_slice(t), rows[b], sems.at[b]).wait()
            sds.append(pltpu.async_copy(
                rows[b], scatter_slice(t), sems.at[NSLOT + b], add=True))
        for b in range(NSLOT):
            sds[b].wait()
            pltpu.async_copy(gather_slice(T + b + NSLOT), rows[b], sems.at[b])

    T0 = (AG_ITERS - 1) * NSLOT
    for b in range(NSLOT):
        t = T0 + b
        pltpu.make_async_copy(gather_slice(t), rows[b], sems.at[b]).wait()
        pltpu.async_copy(
            rows[b], scatter_slice(t), sems.at[NSLOT + b], add=True).wait()
    pltpu.sync_copy(
        yn_hbm.at[sidx_v.at[pl.ds(AG_STEPS * AG_CH, AG_TAIL)]],
        rows0.at[pl.ds(0, AG_TAIL)])
    pltpu.sync_copy(
        rows0.at[pl.ds(0, AG_TAIL)],
        acc.at[didx_v.at[pl.ds(AG_STEPS * AG_CH, AG_TAIL)]], add=True)

    plsc.subcore_barrier()
    pltpu.sync_copy(
        acc.at[pl.ds(sid * ROWS_PER_TILE, ROWS_PER_TILE)],
        out_hbm.at[cid, pl.ds(sid * ROWS_PER_TILE, ROWS_PER_TILE)],
    )


_agg_kernel = pl.kernel(
    _agg_body,
    out_type=jax.ShapeDtypeStruct((NC, N_PAD, D), jnp.float32),
    mesh=_mesh,
    scratch_types=[
        pltpu.VMEM_SHARED((N_PAD, D), jnp.float32),
        pltpu.VMEM((EDGES_PER_TILE,), jnp.int32),
        pltpu.VMEM((EDGES_PER_TILE,), jnp.int32),
        pltpu.VMEM((AG_CH, D), jnp.float32),
        pltpu.VMEM((AG_CH, D), jnp.float32),
        pltpu.VMEM((AG_CH, D), jnp.float32),
        pltpu.SemaphoreType.DMA((2 * NSLOT + 3,)),
    ],
)


def _matmul_body(x_ref, w_ref, o_ref):
    o_ref[...] = lax.dot_general(
        x_ref[...], w_ref[...], (((1,), (1,)), ((), ())),
        preferred_element_type=jnp.float32,
    )


def _norm_body(y_ref, degp_ref, o_ref):
    deg = degp_ref[0] + degp_ref[1]
    inv = lax.rsqrt(deg[:, 0:1] + 1.0)
    o_ref[...] = y_ref[...] * inv


def _final_body(a_ref, yn_ref, b_ref, o_ref):
    o_ref[...] = a_ref[0] + a_ref[1] + yn_ref[...] + b_ref[...]


_RB = 1000
_GRID = N_NODES // _RB


def kernel(x, edge_index, W, b):
    src = edge_index[0]
    dst = edge_index[1]

    y = pl.pallas_call(
        _matmul_body,
        grid=(_GRID,),
        in_specs=[
            pl.BlockSpec((_RB, D), lambda i: (i, 0)),
            pl.BlockSpec((D, D), lambda i: (0, 0)),
        ],
        out_specs=pl.BlockSpec((_RB, D), lambda i: (i, 0)),
        out_shape=jax.ShapeDtypeStruct((N_NODES, D), jnp.float32),
    )(x, W)

    degp = _deg_kernel(src)

    yn = pl.pallas_call(
        _norm_body,
        grid=(_GRID,),
        in_specs=[
            pl.BlockSpec((_RB, D), lambda i: (i, 0)),
            pl.BlockSpec((NC, _RB, DEG_W), lambda i: (0, i, 0)),
        ],
        out_specs=pl.BlockSpec((_RB, D), lambda i: (i, 0)),
        out_shape=jax.ShapeDtypeStruct((N_NODES, D), jnp.float32),
    )(y, degp)

    zeros_rows = jnp.zeros((ROWS_PER_TILE, D), jnp.float32)
    aggp = _agg_kernel(yn, src, dst, zeros_rows)

    out = pl.pallas_call(
        _final_body,
        grid=(_GRID,),
        in_specs=[
            pl.BlockSpec((NC, _RB, D), lambda i: (0, i, 0)),
            pl.BlockSpec((_RB, D), lambda i: (i, 0)),
            pl.BlockSpec((1, D), lambda i: (0, 0)),
        ],
        out_specs=pl.BlockSpec((_RB, D), lambda i: (i, 0)),
        out_shape=jax.ShapeDtypeStruct((N_NODES, D), jnp.float32),
    )(aggp, yn, b.reshape(1, D))

    return out


# speedup vs baseline: 10.7329x; 1.0538x over previous
import functools

import jax
import jax.numpy as jnp
from jax import lax
from jax.experimental import pallas as pl
from jax.experimental.pallas import tpu as pltpu
from jax.experimental.pallas import tpu_sc as plsc

N_NODES = 10000
N_EDGES = 320000
D = 128

NC = 2
NS = 16
NW = NC * NS

EDGES_PER_TILE = N_EDGES // NW
CHUNK = 80
STEPS = EDGES_PER_TILE // CHUNK
ROWS_PER_TILE = 632
N_PAD = NS * ROWS_PER_TILE

DEG_W = 16

_mesh = plsc.VectorSubcoreMesh(
    core_axis_name="c", subcore_axis_name="s", num_cores=NC, num_subcores=NS
)


import dataclasses as _dc

_cp = pltpu.CompilerParams()
if "needs_layout_passes" in pltpu.CompilerParams.__dataclass_fields__:
    _cp = _dc.replace(_cp, needs_layout_passes=False)


def _deg_body(src_hbm, out_hbm, stage, hist_v, idx_v, row_v, acc_v, bcast_v):
    cid = lax.axis_index("c")
    sid = lax.axis_index("s")
    wid = cid * NS + sid

    @pl.loop(0, N_PAD // 16)
    def _(i):
        hist_v[pl.ds(i * 16, 16)] = jnp.zeros((16,), jnp.float32)

    pltpu.sync_copy(src_hbm.at[pl.ds(wid * EDGES_PER_TILE, EDGES_PER_TILE)], idx_v)

    @pl.loop(0, EDGES_PER_TILE // 16)
    def _(i):
        iv = idx_v[pl.ds(i * 16, 16)]
        plsc.addupdate_scatter(hist_v, [iv], jnp.full((16,), 1.0, jnp.float32))

    pltpu.sync_copy(hist_v, stage.at[pl.ds(sid * N_PAD, N_PAD)])
    plsc.subcore_barrier()

    @pl.loop(0, ROWS_PER_TILE // 16)
    def _(k):
        acc_v[pl.ds(k * 16, 16)] = jnp.zeros((16,), jnp.float32)

    @pl.loop(0, NS)
    def _(j):
        pltpu.sync_copy(stage.at[pl.ds(j * N_PAD + sid * ROWS_PER_TILE, ROWS_PER_TILE)], row_v)

        @pl.loop(0, ROWS_PER_TILE // 16)
        def _(k):
            acc_v[pl.ds(k * 16, 16)] = acc_v[pl.ds(k * 16, 16)] + row_v[pl.ds(k * 16, 16)]

    @pl.loop(0, ROWS_PER_TILE)
    def _(i):
        v = plsc.load_gather(acc_v, [jnp.full((16,), i, jnp.int32)])
        bcast_v[i] = v

    pltpu.sync_copy(bcast_v, out_hbm.at[cid, pl.ds(sid * ROWS_PER_TILE, ROWS_PER_TILE)])


_deg_kernel = pl.kernel(
    _deg_body,
    out_type=jax.ShapeDtypeStruct((NC, N_PAD, DEG_W), jnp.float32),
    mesh=_mesh,
    scratch_types=[
        pltpu.VMEM_SHARED((NS * N_PAD,), jnp.float32),
        pltpu.VMEM((N_PAD,), jnp.float32),
        pltpu.VMEM((EDGES_PER_TILE,), jnp.int32),
        pltpu.VMEM((ROWS_PER_TILE,), jnp.float32),
        pltpu.VMEM((ROWS_PER_TILE,), jnp.float32),
        pltpu.VMEM((ROWS_PER_TILE, DEG_W), jnp.float32),
    ],
    compiler_params=_cp,
)


AG_CH = 32
AG_STEPS = 312
AG_TAIL = EDGES_PER_TILE - AG_CH * AG_STEPS
NSLOT = 6
AG_ITERS = AG_STEPS // NSLOT


def _agg_body(yn_hbm, src_hbm, dst_hbm, zeros_hbm, out_hbm,
              acc, sidx_v, didx_v, rows0, rows1, rows2, rows3, rows4, rows5, sems):
    cid = lax.axis_index("c")
    sid = lax.axis_index("s")
    wid = cid * NS + sid
    rows = [rows0, rows1, rows2, rows3, rows4, rows5]
    ebase = wid * EDGES_PER_TILE

    z = pltpu.async_copy(
        zeros_hbm, acc.at[pl.ds(sid * ROWS_PER_TILE, ROWS_PER_TILE)],
        sems.at[2 * NSLOT])
    si = pltpu.async_copy(
        src_hbm.at[pl.ds(ebase, EDGES_PER_TILE)], sidx_v, sems.at[2 * NSLOT + 1])
    di = pltpu.async_copy(
        dst_hbm.at[pl.ds(ebase, EDGES_PER_TILE)], didx_v, sems.at[2 * NSLOT + 2])
    z.wait()
    si.wait()
    di.wait()
    plsc.subcore_barrier()

    def gather_slice(t):
        return yn_hbm.at[sidx_v.at[pl.ds(t * AG_CH, AG_CH)]]

    def scatter_slice(t):
        return acc.at[didx_v.at[pl.ds(t * AG_CH, AG_CH)]]

    for b in range(NSLOT):
        pltpu.async_copy(gather_slice(b), rows[b], sems.at[b])

    @pl.loop(0, AG_ITERS - 1)
    def _(k):
        T = k * NSLOT
        sds = []
        for b in range(NSLOT):
            t = T + b
            pltpu.make_async_copy(gather_slice(t), rows[b], sems.at[b]).wait()
            sds.append(pltpu.async_copy(
                rows[b], scatter_slice(t), sems.at[NSLOT + b], add=True))
        for b in range(NSLOT):
            sds[b].wait()
            pltpu.async_copy(gather_slice(T + b + NSLOT), rows[b], sems.at[b])

    T0 = (AG_ITERS - 1) * NSLOT
    for b in range(NSLOT):
        t = T0 + b
        pltpu.make_async_copy(gather_slice(t), rows[b], sems.at[b]).wait()
        pltpu.async_copy(
            rows[b], scatter_slice(t), sems.at[NSLOT + b], add=True).wait()
    pltpu.sync_copy(
        yn_hbm.at[sidx_v.at[pl.ds(AG_STEPS * AG_CH, AG_TAIL)]],
        rows0.at[pl.ds(0, AG_TAIL)])
    pltpu.sync_copy(
        rows0.at[pl.ds(0, AG_TAIL)],
        acc.at[didx_v.at[pl.ds(AG_STEPS * AG_CH, AG_TAIL)]], add=True)

    plsc.subcore_barrier()
    pltpu.sync_copy(
        acc.at[pl.ds(sid * ROWS_PER_TILE, ROWS_PER_TILE)],
        out_hbm.at[cid, pl.ds(sid * ROWS_PER_TILE, ROWS_PER_TILE)],
    )


_agg_kernel = pl.kernel(
    _agg_body,
    out_type=jax.ShapeDtypeStruct((NC, N_PAD, D), jnp.float32),
    mesh=_mesh,
    scratch_types=[
        pltpu.VMEM_SHARED((N_PAD, D), jnp.float32),
        pltpu.VMEM((EDGES_PER_TILE,), jnp.int32),
        pltpu.VMEM((EDGES_PER_TILE,), jnp.int32),
        pltpu.VMEM((AG_CH, D), jnp.float32),
        pltpu.VMEM((AG_CH, D), jnp.float32),
        pltpu.VMEM((AG_CH, D), jnp.float32),
        pltpu.VMEM((AG_CH, D), jnp.float32),
        pltpu.VMEM((AG_CH, D), jnp.float32),
        pltpu.VMEM((AG_CH, D), jnp.float32),
        pltpu.SemaphoreType.DMA((2 * NSLOT + 3,)),
    ],
)


def _matmul_body(x_ref, w_ref, o_ref):
    o_ref[...] = lax.dot_general(
        x_ref[...], w_ref[...], (((1,), (1,)), ((), ())),
        preferred_element_type=jnp.float32,
    )


def _norm_body(y_ref, degp_ref, o_ref):
    deg = degp_ref[0] + degp_ref[1]
    inv = lax.rsqrt(deg[:, 0:1] + 1.0)
    o_ref[...] = y_ref[...] * inv


def _final_body(a_ref, yn_ref, b_ref, o_ref):
    o_ref[...] = a_ref[0] + a_ref[1] + yn_ref[...] + b_ref[...]


_RB = 1000
_GRID = N_NODES // _RB


def kernel(x, edge_index, W, b):
    src = edge_index[0]
    dst = edge_index[1]

    y = pl.pallas_call(
        _matmul_body,
        grid=(_GRID,),
        in_specs=[
            pl.BlockSpec((_RB, D), lambda i: (i, 0)),
            pl.BlockSpec((D, D), lambda i: (0, 0)),
        ],
        out_specs=pl.BlockSpec((_RB, D), lambda i: (i, 0)),
        out_shape=jax.ShapeDtypeStruct((N_NODES, D), jnp.float32),
    )(x, W)

    degp = _deg_kernel(src)

    yn = pl.pallas_call(
        _norm_body,
        grid=(_GRID,),
        in_specs=[
            pl.BlockSpec((_RB, D), lambda i: (i, 0)),
            pl.BlockSpec((NC, _RB, DEG_W), lambda i: (0, i, 0)),
        ],
        out_specs=pl.BlockSpec((_RB, D), lambda i: (i, 0)),
        out_shape=jax.ShapeDtypeStruct((N_NODES, D), jnp.float32),
    )(y, degp)

    zeros_rows = jnp.zeros((ROWS_PER_TILE, D), jnp.float32)
    aggp = _agg_kernel(yn, src, dst, zeros_rows)

    out = pl.pallas_call(
        _final_body,
        grid=(_GRID,),
        in_specs=[
            pl.BlockSpec((NC, _RB, D), lambda i: (0, i, 0)),
            pl.BlockSpec((_RB, D), lambda i: (i, 0)),
            pl.BlockSpec((1, D), lambda i: (0, 0)),
        ],
        out_specs=pl.BlockSpec((_RB, D), lambda i: (i, 0)),
        out_shape=jax.ShapeDtypeStruct((N_NODES, D), jnp.float32),
    )(aggp, yn, b.reshape(1, D))

    return out
